# Initial kernel scaffold; baseline (speedup 1.0000x reference)
import jax
import jax.numpy as jnp
from jax.experimental import pallas as pl


def kernel(xyz1, xyz2, normal_rebuild, normal_gt):
    raise NotImplementedError("write your pallas kernel here")


# speedup vs baseline: 2.2683x; 2.2683x over previous
import functools

import jax
import jax.numpy as jnp
from jax.experimental import pallas as pl
from jax.experimental.pallas import tpu as pltpu


def _chamfer_block_kernel(lx_ref, ln_ref, rx_ref, rn_ref, oxyz_ref, onrm_ref,
                          cmin_ref, cpx_ref, cpn_ref, *, n_iblocks, inv_count):
    b = pl.program_id(0)
    i = pl.program_id(1)

    lx = lx_ref[0]
    ln = ln_ref[0]
    rx = rx_ref[0]
    rn = rn_ref[0]

    gx = jnp.dot(lx, rx, preferred_element_type=jnp.float32)
    gn = jnp.dot(ln, rn, preferred_element_type=jnp.float32)

    sx1 = jnp.sum(lx * lx, axis=1, keepdims=True)
    sn1 = jnp.sum(ln * ln, axis=1, keepdims=True)
    sx2 = jnp.sum(rx * rx, axis=0, keepdims=True)
    sn2 = jnp.sum(rn * rn, axis=0, keepdims=True)

    dxyz = (sx1 + sx2) - 2.0 * gx
    snn = sn1 + sn2
    d6 = dxyz + (snn - 2.0 * gn)
    dnrm = snn - 2.0 * jnp.abs(gn)

    inf = jnp.float32(jnp.inf)

    m1 = jnp.min(d6, axis=1, keepdims=True)
    mk1 = d6 <= m1
    spx = jnp.sum(jnp.min(jnp.where(mk1, dxyz, inf), axis=1)).reshape(1, 1)
    spn = jnp.sum(jnp.min(jnp.where(mk1, dnrm, inf), axis=1)).reshape(1, 1)

    cm = jnp.min(d6, axis=0, keepdims=True)
    mk2 = d6 <= cm
    bpx = jnp.min(jnp.where(mk2, dxyz, inf), axis=0, keepdims=True)
    bpn = jnp.min(jnp.where(mk2, dnrm, inf), axis=0, keepdims=True)

    @pl.when(i == 0)
    def _init_cols():
        cmin_ref[...] = cm
        cpx_ref[...] = bpx
        cpn_ref[...] = bpn

    @pl.when(i != 0)
    def _fold_cols():
        upd = cm < cmin_ref[...]
        cmin_ref[...] = jnp.where(upd, cm, cmin_ref[...])
        cpx_ref[...] = jnp.where(upd, bpx, cpx_ref[...])
        cpn_ref[...] = jnp.where(upd, bpn, cpn_ref[...])

    first = jnp.logical_and(b == 0, i == 0)
    base_x = jnp.where(first, jnp.zeros((1, 1), jnp.float32), oxyz_ref[...])
    base_n = jnp.where(first, jnp.zeros((1, 1), jnp.float32), onrm_ref[...])
    acc_x = base_x + spx
    acc_n = base_n + spn
    oxyz_ref[...] = acc_x
    onrm_ref[...] = acc_n

    @pl.when(i == n_iblocks - 1)
    def _finish_batch():
        tot_x = acc_x + jnp.sum(cpx_ref[...]).reshape(1, 1)
        tot_n = acc_n + jnp.sum(cpn_ref[...]).reshape(1, 1)
        scale = jnp.where(b == pl.num_programs(0) - 1, inv_count, 1.0)
        oxyz_ref[...] = tot_x * scale
        onrm_ref[...] = tot_n * scale


def _normalize(x, eps=1e-12):
    n = jnp.sqrt(jnp.sum(x * x, axis=2, keepdims=True))
    return x / jnp.maximum(n, eps)


def _pad_feat(x):
    return jnp.pad(x, ((0, 0), (0, 0), (0, 5)))


@jax.jit
def kernel(xyz1, xyz2, normal_rebuild, normal_gt):
    B, N1, _ = xyz1.shape
    N2 = xyz2.shape[1]

    nr = _normalize(normal_rebuild)
    ng = _normalize(normal_gt)

    lx = _pad_feat(xyz1)
    ln = _pad_feat(nr)
    rx = jnp.transpose(_pad_feat(xyz2), (0, 2, 1))
    rn = jnp.transpose(_pad_feat(ng), (0, 2, 1))

    TI = 512 if N1 % 512 == 0 else N1
    n_iblocks = N1 // TI
    inv_count = 1.0 / (B * N1)

    grid = (B, n_iblocks)
    out_xyz, out_nrm = pl.pallas_call(
        functools.partial(_chamfer_block_kernel, n_iblocks=n_iblocks,
                          inv_count=inv_count),
        grid=grid,
        in_specs=[
            pl.BlockSpec((1, TI, 8), lambda b, i: (b, i, 0)),
            pl.BlockSpec((1, TI, 8), lambda b, i: (b, i, 0)),
            pl.BlockSpec((1, 8, N2), lambda b, i: (b, 0, 0)),
            pl.BlockSpec((1, 8, N2), lambda b, i: (b, 0, 0)),
        ],
        out_specs=[
            pl.BlockSpec((1, 1), lambda b, i: (0, 0)),
            pl.BlockSpec((1, 1), lambda b, i: (0, 0)),
        ],
        out_shape=[
            jax.ShapeDtypeStruct((1, 1), jnp.float32),
            jax.ShapeDtypeStruct((1, 1), jnp.float32),
        ],
        scratch_shapes=[
            pltpu.VMEM((1, N2), jnp.float32),
            pltpu.VMEM((1, N2), jnp.float32),
            pltpu.VMEM((1, N2), jnp.float32),
        ],
    )(lx, ln, rx, rn)

    return (out_xyz[0, 0], out_nrm[0, 0])


# speedup vs baseline: 4.4015x; 1.9404x over previous
import functools

import jax
import jax.numpy as jnp
from jax.experimental import pallas as pl
from jax.experimental.pallas import tpu as pltpu


def _chamfer_block_kernel(l_ref, rd_ref, rn_ref, sn1_ref, oxyz_ref, onrm_ref,
                          cmin_ref, cg_ref, *, n_iblocks, inv_count):
    b = pl.program_id(0)
    i = pl.program_id(1)

    L = l_ref[0]
    Rd = rd_ref[0]
    Rn = rn_ref[0]
    sn1 = sn1_ref[0]

    d6 = jnp.dot(L, Rd, preferred_element_type=jnp.float32)
    gn = jnp.dot(L, Rn, preferred_element_type=jnp.float32)

    inf = jnp.float32(jnp.inf)

    m1 = jnp.min(d6, axis=1, keepdims=True)
    mk1 = d6 <= m1
    g1 = jnp.min(jnp.where(mk1, gn, inf), axis=1, keepdims=True)
    snn1 = sn1 + 1.0
    spx = jnp.sum(m1 - snn1 + 2.0 * g1).reshape(1, 1)
    spn = jnp.sum(snn1 - 2.0 * jnp.abs(g1)).reshape(1, 1)

    cm = jnp.min(d6, axis=0, keepdims=True)
    mk2 = d6 <= cm
    g2 = jnp.min(jnp.where(mk2, gn, inf), axis=0, keepdims=True)

    @pl.when(i == 0)
    def _init_cols():
        cmin_ref[...] = cm
        cg_ref[...] = g2

    @pl.when(i != 0)
    def _fold_cols():
        upd = cm < cmin_ref[...]
        cmin_ref[...] = jnp.where(upd, cm, cmin_ref[...])
        cg_ref[...] = jnp.where(upd, g2, cg_ref[...])

    first = jnp.logical_and(b == 0, i == 0)
    base_x = jnp.where(first, jnp.zeros((1, 1), jnp.float32), oxyz_ref[...])
    base_n = jnp.where(first, jnp.zeros((1, 1), jnp.float32), onrm_ref[...])
    acc_x = base_x + spx
    acc_n = base_n + spn
    oxyz_ref[...] = acc_x
    onrm_ref[...] = acc_n

    @pl.when(i == n_iblocks - 1)
    def _finish_batch():
        cmin = cmin_ref[...]
        cg = cg_ref[...]
        colpx = cmin - 2.0 + 2.0 * cg
        colpn = 2.0 - 2.0 * jnp.abs(cg)
        tot_x = acc_x + jnp.sum(colpx).reshape(1, 1)
        tot_n = acc_n + jnp.sum(colpn).reshape(1, 1)
        scale = jnp.where(b == pl.num_programs(0) - 1, inv_count, 1.0)
        oxyz_ref[...] = tot_x * scale
        onrm_ref[...] = tot_n * scale


def _normalize(x, eps=1e-12):
    n = jnp.sqrt(jnp.sum(x * x, axis=2, keepdims=True))
    return x / jnp.maximum(n, eps)


@jax.jit
def kernel(xyz1, xyz2, normal_rebuild, normal_gt):
    B, N1, _ = xyz1.shape
    N2 = xyz2.shape[1]

    nr = _normalize(normal_rebuild)
    ng = _normalize(normal_gt)

    sx1 = jnp.sum(xyz1 * xyz1, axis=2, keepdims=True)
    sn1 = jnp.sum(nr * nr, axis=2, keepdims=True)
    sx2 = jnp.sum(xyz2 * xyz2, axis=2, keepdims=True)
    sn2 = jnp.sum(ng * ng, axis=2, keepdims=True)

    ones1 = jnp.ones((B, N1, 1), jnp.float32)
    L = jnp.concatenate([xyz1, nr, sx1 + sn1, ones1], axis=2)
    Rd = jnp.concatenate([-2.0 * xyz2, -2.0 * ng, ones1[:, :N2],
                          sx2 + sn2], axis=2)
    Rd = jnp.transpose(Rd, (0, 2, 1))
    zeros2 = jnp.zeros((B, N2, 3), jnp.float32)
    Rn = jnp.concatenate([zeros2, ng, jnp.zeros((B, N2, 2), jnp.float32)],
                         axis=2)
    Rn = jnp.transpose(Rn, (0, 2, 1))

    TI = 512 if N1 % 512 == 0 else N1
    n_iblocks = N1 // TI
    inv_count = 1.0 / (B * N1)

    grid = (B, n_iblocks)
    out_xyz, out_nrm = pl.pallas_call(
        functools.partial(_chamfer_block_kernel, n_iblocks=n_iblocks,
                          inv_count=inv_count),
        grid=grid,
        in_specs=[
            pl.BlockSpec((1, TI, 8), lambda b, i: (b, i, 0)),
            pl.BlockSpec((1, 8, N2), lambda b, i: (b, 0, 0)),
            pl.BlockSpec((1, 8, N2), lambda b, i: (b, 0, 0)),
            pl.BlockSpec((1, TI, 1), lambda b, i: (b, i, 0)),
        ],
        out_specs=[
            pl.BlockSpec((1, 1), lambda b, i: (0, 0)),
            pl.BlockSpec((1, 1), lambda b, i: (0, 0)),
        ],
        out_shape=[
            jax.ShapeDtypeStruct((1, 1), jnp.float32),
            jax.ShapeDtypeStruct((1, 1), jnp.float32),
        ],
        scratch_shapes=[
            pltpu.VMEM((1, N2), jnp.float32),
            pltpu.VMEM((1, N2), jnp.float32),
        ],
    )(L, Rd, Rn, sn1)

    return (out_xyz[0, 0], out_nrm[0, 0])
